# baseline (speedup 1.0000x reference)
import functools

import jax
import jax.numpy as jnp
from jax.experimental import pallas as pl
from jax.experimental.pallas import tpu as pltpu

REP = 128
NE = 320000
NCYC = 5000
SIZES = (3, 4, 5, 6, 7, 8)
TOTC = NCYC * len(SIZES)
EPS = 1e-5


def _cyc_pass1_body(cr_ref, pooled_ref, a_ref, m_ref, stat_ref):
    i = pl.program_id(0)
    y1 = (jnp.dot(cr_ref[...], a_ref[...], preferred_element_type=jnp.float32)
          + jnp.dot(pooled_ref[...], m_ref[0], preferred_element_type=jnp.float32))

    @pl.when(i == 0)
    def _():
        stat_ref[...] = jnp.zeros_like(stat_ref)

    stat_ref[0, :] += jnp.sum(y1, axis=0)
    stat_ref[1, :] += jnp.sum(y1 * y1, axis=0)


def _cyc_pass2_body(cr_ref, pooled_ref, a_ref, m_ref, sc1_ref, w2_ref,
                    y2_ref, stat_ref):
    i = pl.program_id(0)
    y1 = (jnp.dot(cr_ref[...], a_ref[...], preferred_element_type=jnp.float32)
          + jnp.dot(pooled_ref[...], m_ref[0], preferred_element_type=jnp.float32))
    h = jnp.maximum(y1 * sc1_ref[0, :] + sc1_ref[1, :], 0.0)
    y2 = jnp.dot(h, w2_ref[...], preferred_element_type=jnp.float32)
    y2_ref[...] = y2

    @pl.when(i == 0)
    def _():
        stat_ref[...] = jnp.zeros_like(stat_ref)

    stat_ref[0, :] += jnp.sum(y2, axis=0)
    stat_ref[1, :] += jnp.sum(y2 * y2, axis=0)


def _norm_body(y_ref, sc_ref, out_ref):
    out_ref[...] = jnp.maximum(y_ref[...] * sc_ref[0, :] + sc_ref[1, :], 0.0)


def _edge_pass1_body(er_ref, c2e_ref, w1a_ref, w1b_ref, stat_ref):
    i = pl.program_id(0)
    y1 = (jnp.dot(er_ref[...], w1a_ref[...], preferred_element_type=jnp.float32)
          + jnp.dot(c2e_ref[...], w1b_ref[...], preferred_element_type=jnp.float32))

    @pl.when(i == 0)
    def _():
        stat_ref[...] = jnp.zeros_like(stat_ref)

    stat_ref[0, :] += jnp.sum(y1, axis=0)
    stat_ref[1, :] += jnp.sum(y1 * y1, axis=0)


def _edge_pass2_body(er_ref, c2e_ref, w1a_ref, w1b_ref, sc1_ref, w2_ref,
                     y2_ref, stat_ref):
    i = pl.program_id(0)
    y1 = (jnp.dot(er_ref[...], w1a_ref[...], preferred_element_type=jnp.float32)
          + jnp.dot(c2e_ref[...], w1b_ref[...], preferred_element_type=jnp.float32))
    h = jnp.maximum(y1 * sc1_ref[0, :] + sc1_ref[1, :], 0.0)
    y2 = jnp.dot(h, w2_ref[...], preferred_element_type=jnp.float32)
    y2_ref[...] = y2

    @pl.when(i == 0)
    def _():
        stat_ref[...] = jnp.zeros_like(stat_ref)

    stat_ref[0, :] += jnp.sum(y2, axis=0)
    stat_ref[1, :] += jnp.sum(y2 * y2, axis=0)


def _scale_shift(stats, n, g, b):
    mean = stats[0] / n
    var = stats[1] / n - mean * mean
    scale = g / jnp.sqrt(var + EPS)
    shift = b - mean * scale
    return jnp.stack([scale, shift])


_CB = 1000
_CG = TOTC // _CB
_EB = 2000
_EG = NE // _EB


def _row_spec(blk, width):
    return pl.BlockSpec((blk, width), lambda i: (i, 0))


def _full_spec(shape):
    return pl.BlockSpec(shape, lambda i: tuple(0 for _ in shape))


def _cycle_mlp(cycle_rep, pooled, A, Ms, cyc_g1, cyc_b1, cyc_W2, cyc_g2, cyc_b2):
    m_spec = pl.BlockSpec((1, REP, 2 * REP), lambda i: (i // (NCYC // _CB), 0, 0))
    stats1 = pl.pallas_call(
        _cyc_pass1_body,
        grid=(_CG,),
        in_specs=[_row_spec(_CB, REP), _row_spec(_CB, REP),
                  _full_spec((REP, 2 * REP)), m_spec],
        out_specs=_full_spec((2, 2 * REP)),
        out_shape=jax.ShapeDtypeStruct((2, 2 * REP), jnp.float32),
    )(cycle_rep, pooled, A, Ms)
    sc1 = _scale_shift(stats1, TOTC, cyc_g1, cyc_b1)

    y2, stats2 = pl.pallas_call(
        _cyc_pass2_body,
        grid=(_CG,),
        in_specs=[_row_spec(_CB, REP), _row_spec(_CB, REP),
                  _full_spec((REP, 2 * REP)), m_spec,
                  _full_spec((2, 2 * REP)), _full_spec((2 * REP, REP))],
        out_specs=[_row_spec(_CB, REP), _full_spec((2, REP))],
        out_shape=[jax.ShapeDtypeStruct((TOTC, REP), jnp.float32),
                   jax.ShapeDtypeStruct((2, REP), jnp.float32)],
    )(cycle_rep, pooled, A, Ms, sc1, cyc_W2)
    sc2 = _scale_shift(stats2, TOTC, cyc_g2, cyc_b2)

    cycle_out = pl.pallas_call(
        _norm_body,
        grid=(_CG,),
        in_specs=[_row_spec(_CB, REP), _full_spec((2, REP))],
        out_specs=_row_spec(_CB, REP),
        out_shape=jax.ShapeDtypeStruct((TOTC, REP), jnp.float32),
    )(y2, sc2)
    return cycle_out


def _edge_mlp(edge_rep, c2e, W1a, W1b, edge_g1, edge_b1, edge_W2, edge_g2, edge_b2):
    stats1 = pl.pallas_call(
        _edge_pass1_body,
        grid=(_EG,),
        in_specs=[_row_spec(_EB, REP), _row_spec(_EB, REP),
                  _full_spec((REP, 2 * REP)), _full_spec((REP, 2 * REP))],
        out_specs=_full_spec((2, 2 * REP)),
        out_shape=jax.ShapeDtypeStruct((2, 2 * REP), jnp.float32),
    )(edge_rep, c2e, W1a, W1b)
    sc1 = _scale_shift(stats1, NE, edge_g1, edge_b1)

    y2, stats2 = pl.pallas_call(
        _edge_pass2_body,
        grid=(_EG,),
        in_specs=[_row_spec(_EB, REP), _row_spec(_EB, REP),
                  _full_spec((REP, 2 * REP)), _full_spec((REP, 2 * REP)),
                  _full_spec((2, 2 * REP)), _full_spec((2 * REP, REP))],
        out_specs=[_row_spec(_EB, REP), _full_spec((2, REP))],
        out_shape=[jax.ShapeDtypeStruct((NE, REP), jnp.float32),
                   jax.ShapeDtypeStruct((2, REP), jnp.float32)],
    )(edge_rep, c2e, W1a, W1b, sc1, edge_W2)
    sc2 = _scale_shift(stats2, NE, edge_g2, edge_b2)

    edge_out = pl.pallas_call(
        _norm_body,
        grid=(_EG,),
        in_specs=[_row_spec(_EB, REP), _full_spec((2, REP))],
        out_specs=_row_spec(_EB, REP),
        out_shape=jax.ShapeDtypeStruct((NE, REP), jnp.float32),
    )(y2, sc2)
    return edge_out


def _gather_pooled(edge_rep, idxs):
    per_size = [jnp.take(edge_rep, idx, axis=0).sum(axis=1) for idx in idxs]
    return jnp.concatenate(per_size, axis=0)


def _scatter_c2e(cycle_out, idxs):
    c2e = jnp.zeros((NE, REP), dtype=cycle_out.dtype)
    off = 0
    for idx in idxs:
        n, s = idx.shape
        co = cycle_out[off:off + n]
        c2e = c2e.at[idx.reshape(-1)].add(jnp.repeat(co, s, axis=0))
        off += n
    return c2e


def kernel(edge_rep, cycle_rep, cyc3_idx, cyc4_idx, cyc5_idx, cyc6_idx, cyc7_idx, cyc8_idx,
           aut_W, cyc_W1, cyc_g1, cyc_b1, cyc_W2, cyc_g2, cyc_b2,
           edge_W1, edge_g1, edge_b1, edge_W2, edge_g2, edge_b2):
    idxs = [cyc3_idx, cyc4_idx, cyc5_idx, cyc6_idx, cyc7_idx, cyc8_idx]

    A = cyc_W1[:REP]
    Bs = cyc_W1[REP:].reshape(2, REP, 2 * REP)
    Ms = jnp.einsum('cikl,clo->iko', aut_W, Bs)

    pooled = _gather_pooled(edge_rep, idxs)
    cycle_out = _cycle_mlp(cycle_rep, pooled, A, Ms, cyc_g1, cyc_b1,
                           cyc_W2, cyc_g2, cyc_b2)

    c2e = _scatter_c2e(cycle_out, idxs)
    W1a = edge_W1[:REP]
    W1b = edge_W1[REP:]
    edge_out = _edge_mlp(edge_rep, c2e, W1a, W1b, edge_g1, edge_b1,
                         edge_W2, edge_g2, edge_b2)
    return edge_out, cycle_out
